# Initial kernel scaffold; baseline (speedup 1.0000x reference)
import jax
import jax.numpy as jnp
from jax.experimental import pallas as pl


def kernel(features, neighbors, importance_weights, W, b, gamma, beta):
    raise NotImplementedError("write your pallas kernel here")


# speedup vs baseline: 1.8038x; 1.8038x over previous
import functools

import jax
import jax.numpy as jnp
from jax import lax
from jax.experimental import pallas as pl
from jax.experimental.pallas import tpu as pltpu
from jax.experimental.pallas import tpu_sc as plsc

N = 10000
K = 32
D = 128

NW = 32
NPAD = 10240
NPW = NPAD // NW
C = 4
G = C * K
NSTEPS = NPW // C
IDX_ROWS = NPAD * K // G


def _sc_aggregate(features, idx2d, weights):
    mesh = plsc.VectorSubcoreMesh(core_axis_name="c", subcore_axis_name="s")

    @functools.partial(
        pl.kernel,
        out_type=jax.ShapeDtypeStruct((NPAD, D), jnp.float32),
        mesh=mesh,
        scratch_types=[
            pltpu.VMEM((NSTEPS, G), jnp.int32),
            pltpu.VMEM((NPW, K), jnp.float32),
            pltpu.VMEM((G, D), jnp.float32),
            pltpu.VMEM((G, D), jnp.float32),
            pltpu.VMEM((NPW, D), jnp.float32),
            pltpu.SemaphoreType.DMA,
            pltpu.SemaphoreType.DMA,
        ],
    )
    def agg_kernel(feat_hbm, idx_hbm, w_hbm, out_hbm,
                   idx_v, w_v, rows0, rows1, out_v, sem0, sem1):
        wid = lax.axis_index("s") * 2 + lax.axis_index("c")

        pltpu.sync_copy(idx_hbm.at[pl.ds(wid * NSTEPS, NSTEPS)], idx_v)
        pltpu.make_async_copy(
            feat_hbm.at[idx_v.at[0]], rows0, sem0).start()
        pltpu.make_async_copy(
            feat_hbm.at[idx_v.at[1]], rows1, sem1).start()

        pltpu.sync_copy(w_hbm.at[pl.ds(wid * NPW, NPW)], w_v)

        def compute(step, rows):
            def node_body(nl, carry):
                ln = step * C + nl
                wv0 = w_v[ln, pl.ds(0, 16)]
                wv1 = w_v[ln, pl.ds(16, 16)]
                acc = [jnp.zeros((16,), jnp.float32) for _ in range(D // 16)]
                for k in range(K):
                    ws = wv0[k] if k < 16 else wv1[k - 16]
                    r = nl * K + k
                    for d in range(D // 16):
                        acc[d] = acc[d] + ws * rows[r, pl.ds(d * 16, 16)]
                for d in range(D // 16):
                    out_v[ln, pl.ds(d * 16, 16)] = acc[d]
                return carry
            lax.fori_loop(0, C, node_body, 0)

        def main_body(g, carry):
            step = g * 2
            pltpu.make_async_copy(
                feat_hbm.at[idx_v.at[step]], rows0, sem0).wait()
            compute(step, rows0)

            @pl.when(step + 2 < NSTEPS)
            def _():
                pltpu.make_async_copy(
                    feat_hbm.at[idx_v.at[step + 2]], rows0, sem0).start()

            pltpu.make_async_copy(
                feat_hbm.at[idx_v.at[step + 1]], rows1, sem1).wait()
            compute(step + 1, rows1)

            @pl.when(step + 3 < NSTEPS)
            def _():
                pltpu.make_async_copy(
                    feat_hbm.at[idx_v.at[step + 3]], rows1, sem1).start()
            return carry
        lax.fori_loop(0, NSTEPS // 2, main_body, 0)

        pltpu.sync_copy(out_v, out_hbm.at[pl.ds(wid * NPW, NPW)])

    return agg_kernel(features, idx2d, weights)


def _tc_normalize_weights(w2d, Q):
    BR = 512

    def body(w_ref, q_ref, o_ref):
        w = w_ref[...]
        s = lax.dot_general(w, q_ref[...], (((1,), (0,)), ((), ())),
                            preferred_element_type=jnp.float32)
        z = s == 0.0
        safe = jnp.where(z, 1.0, s)
        o_ref[...] = jnp.where(z, jnp.float32(1.0 / K), w / safe)

    return pl.pallas_call(
        body,
        grid=(IDX_ROWS // BR,),
        in_specs=[
            pl.BlockSpec((BR, G), lambda i: (i, 0)),
            pl.BlockSpec((G, G), lambda i: (0, 0)),
        ],
        out_specs=pl.BlockSpec((BR, G), lambda i: (i, 0)),
        out_shape=jax.ShapeDtypeStruct((IDX_ROWS, G), jnp.float32),
    )(w2d, Q)


def _tc_linear_layernorm(agg, W, b, gamma, beta):
    BR = 1024

    def body(x_ref, w_ref, b_ref, g_ref, be_ref, o_ref):
        x = x_ref[...]
        y = lax.dot_general(x, w_ref[...], (((1,), (1,)), ((), ())),
                            preferred_element_type=jnp.float32)
        y = y + b_ref[...]
        m = jnp.mean(y, axis=-1, keepdims=True)
        dlt = y - m
        var = jnp.mean(dlt * dlt, axis=-1, keepdims=True)
        o_ref[...] = (dlt * lax.rsqrt(var + 1e-5)) * g_ref[...] + be_ref[...]

    return pl.pallas_call(
        body,
        grid=(NPAD // BR,),
        in_specs=[
            pl.BlockSpec((BR, D), lambda i: (i, 0)),
            pl.BlockSpec((D, D), lambda i: (0, 0)),
            pl.BlockSpec((1, D), lambda i: (0, 0)),
            pl.BlockSpec((1, D), lambda i: (0, 0)),
            pl.BlockSpec((1, D), lambda i: (0, 0)),
        ],
        out_specs=pl.BlockSpec((BR, D), lambda i: (i, 0)),
        out_shape=jax.ShapeDtypeStruct((NPAD, D), jnp.float32),
    )(agg, W, b.reshape(1, D), gamma.reshape(1, D), beta.reshape(1, D))


def kernel(features, neighbors, importance_weights, W, b, gamma, beta):
    idx = neighbors.astype(jnp.int32)
    pad = NPAD - N
    idx_p = jnp.pad(idx, ((0, pad), (0, 0)))
    w_p = jnp.pad(importance_weights, ((0, pad), (0, 0)), constant_values=1.0)
    idx2d = idx_p.reshape(IDX_ROWS, G)
    blk = jnp.arange(G, dtype=jnp.int32) // K
    Q = (blk[:, None] == blk[None, :]).astype(jnp.float32)
    w_norm = _tc_normalize_weights(w_p.reshape(IDX_ROWS, G), Q)
    agg = _sc_aggregate(features, idx2d, w_norm.reshape(NPAD, K))
    out = _tc_linear_layernorm(agg, W, b, gamma, beta)
    return out[:N]


# speedup vs baseline: 1.8162x; 1.0069x over previous
import functools

import jax
import jax.numpy as jnp
from jax import lax
from jax.experimental import pallas as pl
from jax.experimental.pallas import tpu as pltpu
from jax.experimental.pallas import tpu_sc as plsc

N = 10000
K = 32
D = 128

NW = 32
NPAD = 10240
NPW = NPAD // NW
C = 4
G = C * K
NSTEPS = NPW // C
IDX_ROWS = NPAD * K // G


def _sc_aggregate(features, idx2d, weights):
    mesh = plsc.VectorSubcoreMesh(core_axis_name="c", subcore_axis_name="s")

    NBUF = 4

    @functools.partial(
        pl.kernel,
        out_type=jax.ShapeDtypeStruct((NPAD, D), jnp.float32),
        mesh=mesh,
        scratch_types=[
            pltpu.VMEM((NSTEPS, G), jnp.int32),
            pltpu.VMEM((NPW, K), jnp.float32),
            pltpu.VMEM((NBUF, G, D), jnp.float32),
            pltpu.VMEM((2, C, D), jnp.float32),
            pltpu.SemaphoreType.DMA,
            pltpu.SemaphoreType.DMA,
            pltpu.SemaphoreType.DMA,
            pltpu.SemaphoreType.DMA,
            pltpu.SemaphoreType.DMA,
            pltpu.SemaphoreType.DMA,
        ],
    )
    def agg_kernel(feat_hbm, idx_hbm, w_hbm, out_hbm,
                   idx_v, w_v, rows_v, ob_v,
                   gs0, gs1, gs2, gs3, os0, os1):
        gsems = [gs0, gs1, gs2, gs3]
        osems = [os0, os1]
        wid = lax.axis_index("s") * 2 + lax.axis_index("c")
        obase = wid * NPW

        def gather(step, rb):
            return pltpu.make_async_copy(
                feat_hbm.at[idx_v.at[step]], rows_v.at[rb], gsems[rb])

        def out_dma(step, ob):
            return pltpu.make_async_copy(
                ob_v.at[ob], out_hbm.at[pl.ds(obase + step * C, C)],
                osems[ob])

        pltpu.sync_copy(idx_hbm.at[pl.ds(wid * NSTEPS, NSTEPS)], idx_v)
        for b in range(NBUF):
            gather(b, b).start()
        pltpu.sync_copy(w_hbm.at[pl.ds(wid * NPW, NPW)], w_v)

        def compute(step, rb, ob):
            def node_body(nl, carry):
                ln = step * C + nl
                wv0 = w_v[ln, pl.ds(0, 16)]
                wv1 = w_v[ln, pl.ds(16, 16)]
                acc = [jnp.zeros((16,), jnp.float32) for _ in range(D // 16)]
                for k in range(K):
                    ws = wv0[k] if k < 16 else wv1[k - 16]
                    r = nl * K + k
                    for d in range(D // 16):
                        acc[d] = acc[d] + ws * rows_v[rb, r, pl.ds(d * 16, 16)]
                for d in range(D // 16):
                    ob_v[ob, nl, pl.ds(d * 16, 16)] = acc[d]
                return carry
            lax.fori_loop(0, C, node_body, 0)

        def step_work(step, rb, ob):
            gather(step, rb).wait()

            @pl.when(step >= 2)
            def _():
                out_dma(step - 2, ob).wait()

            compute(step, rb, ob)

            @pl.when(step + NBUF < NSTEPS)
            def _():
                gather(step + NBUF, rb).start()

            out_dma(step, ob).start()

        def main_body(g, carry):
            base = g * NBUF
            for b in range(NBUF):
                step_work(base + b, b, b % 2)
            return carry
        lax.fori_loop(0, NSTEPS // NBUF, main_body, 0)

        out_dma(NSTEPS - 2, (NSTEPS - 2) % 2).wait()
        out_dma(NSTEPS - 1, (NSTEPS - 1) % 2).wait()

    return agg_kernel(features, idx2d, weights)


def _tc_normalize_weights(w2d, Q):
    BR = 512

    def body(w_ref, q_ref, o_ref):
        w = w_ref[...]
        s = lax.dot_general(w, q_ref[...], (((1,), (0,)), ((), ())),
                            preferred_element_type=jnp.float32)
        z = s == 0.0
        safe = jnp.where(z, 1.0, s)
        o_ref[...] = jnp.where(z, jnp.float32(1.0 / K), w / safe)

    return pl.pallas_call(
        body,
        grid=(IDX_ROWS // BR,),
        in_specs=[
            pl.BlockSpec((BR, G), lambda i: (i, 0)),
            pl.BlockSpec((G, G), lambda i: (0, 0)),
        ],
        out_specs=pl.BlockSpec((BR, G), lambda i: (i, 0)),
        out_shape=jax.ShapeDtypeStruct((IDX_ROWS, G), jnp.float32),
    )(w2d, Q)


def _tc_linear_layernorm(agg, W, b, gamma, beta):
    BR = 1024

    def body(x_ref, w_ref, b_ref, g_ref, be_ref, o_ref):
        x = x_ref[...]
        y = lax.dot_general(x, w_ref[...], (((1,), (1,)), ((), ())),
                            preferred_element_type=jnp.float32)
        y = y + b_ref[...]
        m = jnp.mean(y, axis=-1, keepdims=True)
        dlt = y - m
        var = jnp.mean(dlt * dlt, axis=-1, keepdims=True)
        o_ref[...] = (dlt * lax.rsqrt(var + 1e-5)) * g_ref[...] + be_ref[...]

    return pl.pallas_call(
        body,
        grid=(NPAD // BR,),
        in_specs=[
            pl.BlockSpec((BR, D), lambda i: (i, 0)),
            pl.BlockSpec((D, D), lambda i: (0, 0)),
            pl.BlockSpec((1, D), lambda i: (0, 0)),
            pl.BlockSpec((1, D), lambda i: (0, 0)),
            pl.BlockSpec((1, D), lambda i: (0, 0)),
        ],
        out_specs=pl.BlockSpec((BR, D), lambda i: (i, 0)),
        out_shape=jax.ShapeDtypeStruct((NPAD, D), jnp.float32),
    )(agg, W, b.reshape(1, D), gamma.reshape(1, D), beta.reshape(1, D))


def kernel(features, neighbors, importance_weights, W, b, gamma, beta):
    idx = neighbors.astype(jnp.int32)
    pad = NPAD - N
    idx_p = jnp.pad(idx, ((0, pad), (0, 0)))
    w_p = jnp.pad(importance_weights, ((0, pad), (0, 0)), constant_values=1.0)
    idx2d = idx_p.reshape(IDX_ROWS, G)
    blk = jnp.arange(G, dtype=jnp.int32) // K
    Q = (blk[:, None] == blk[None, :]).astype(jnp.float32)
    w_norm = _tc_normalize_weights(w_p.reshape(IDX_ROWS, G), Q)
    agg = _sc_aggregate(features, idx2d, w_norm.reshape(NPAD, K))
    out = _tc_linear_layernorm(agg, W, b, gamma, beta)
    return out[:N]


# speedup vs baseline: 4.7338x; 2.6064x over previous
import functools

import jax
import jax.numpy as jnp
from jax import lax
from jax.experimental import pallas as pl
from jax.experimental.pallas import tpu as pltpu
from jax.experimental.pallas import tpu_sc as plsc

N = 10000
K = 32
D = 128

NW = 32
NPAD = 10240
NPW = NPAD // NW
G = K
NSTEPS = NPW
W2D_ROWS = NPAD * K // 128


def _sc_aggregate(features, idx_flat, w_flat):
    mesh = plsc.VectorSubcoreMesh(core_axis_name="c", subcore_axis_name="s")

    @functools.partial(
        pl.kernel,
        out_type=jax.ShapeDtypeStruct((NPAD, D), jnp.float32),
        mesh=mesh,
        scratch_types=[
            pltpu.VMEM((2, K), jnp.int32),
            pltpu.VMEM((2, K), jnp.float32),
            pltpu.VMEM((2, G, D), jnp.float32),
            pltpu.VMEM((2, 1, D), jnp.float32),
            pltpu.VMEM_SHARED((N, D), jnp.float32),
            pltpu.SemaphoreType.DMA,
            pltpu.SemaphoreType.DMA,
            pltpu.SemaphoreType.DMA,
            pltpu.SemaphoreType.DMA,
            pltpu.SemaphoreType.DMA,
            pltpu.SemaphoreType.DMA,
            pltpu.SemaphoreType.DMA,
            pltpu.SemaphoreType.DMA,
        ],
    )
    def agg_kernel(feat_hbm, idx_hbm, w_hbm, out_hbm,
                   ich_v, wch_v, rows_v, ob_v, feat_sp,
                   gs0, gs1, os0, os1, is0, is1, ws0, ws1):
        gsems = [gs0, gs1]
        osems = [os0, os1]
        isems = [is0, is1]
        wsems = [ws0, ws1]
        sid = lax.axis_index("s")
        wid = sid * 2 + lax.axis_index("c")
        obase = wid * NPW
        fbase = wid * NPW * K

        def ich_dma(step, j):
            return pltpu.make_async_copy(
                idx_hbm.at[pl.ds(fbase + step * K, K)], ich_v.at[j],
                isems[j])

        def wch_dma(step, j):
            return pltpu.make_async_copy(
                w_hbm.at[pl.ds(fbase + step * K, K)], wch_v.at[j], wsems[j])

        def gather(j):
            return pltpu.make_async_copy(
                feat_sp.at[ich_v.at[j]], rows_v.at[j], gsems[j])

        def out_dma(step, j):
            return pltpu.make_async_copy(
                ob_v.at[j], out_hbm.at[pl.ds(obase + step, 1)], osems[j])

        srows = 632
        soff = pl.multiple_of(jnp.minimum(sid * srows, N - srows), 8)
        pltpu.sync_copy(feat_hbm.at[pl.ds(soff, srows)],
                        feat_sp.at[pl.ds(soff, srows)])
        for j in range(2):
            pltpu.sync_copy(idx_hbm.at[pl.ds(fbase + j * K, K)], ich_v.at[j])
            pltpu.sync_copy(w_hbm.at[pl.ds(fbase + j * K, K)], wch_v.at[j])
        plsc.subcore_barrier()
        for j in range(2):
            gather(j).start()

        def compute(j):
            wv0 = wch_v[j, pl.ds(0, 16)]
            wv1 = wch_v[j, pl.ds(16, 16)]
            acc = [jnp.zeros((16,), jnp.float32) for _ in range(D // 16)]
            for k in range(K):
                ws = wv0[k] if k < 16 else wv1[k - 16]
                for d in range(D // 16):
                    acc[d] = acc[d] + ws * rows_v[j, k, pl.ds(d * 16, 16)]
            for d in range(D // 16):
                ob_v[j, 0, pl.ds(d * 16, 16)] = acc[d]

        def step_work(step, j):
            gather(j).wait()

            @pl.when(step + 2 < NSTEPS)
            def _():
                ich_dma(step + 2, j).start()

            @pl.when(step >= 2)
            def _():
                wch_dma(step, j).wait()
                out_dma(step - 2, j).wait()

            compute(j)
            out_dma(step, j).start()

            @pl.when(step + 2 < NSTEPS)
            def _():
                wch_dma(step + 2, j).start()
                ich_dma(step + 2, j).wait()
                gather(j).start()

        def main_body(g, carry):
            for j in range(2):
                step_work(g * 2 + j, j)
            return carry
        lax.fori_loop(0, NSTEPS // 2, main_body, 0)

        out_dma(NSTEPS - 2, 0).wait()
        out_dma(NSTEPS - 1, 1).wait()

    return agg_kernel(features, idx_flat, w_flat)


def _tc_normalize_weights(w2d, Q):
    BR = 512

    def body(w_ref, q_ref, o_ref):
        w = w_ref[...]
        s = lax.dot_general(w, q_ref[...], (((1,), (0,)), ((), ())),
                            preferred_element_type=jnp.float32)
        z = s == 0.0
        safe = jnp.where(z, 1.0, s)
        o_ref[...] = jnp.where(z, jnp.float32(1.0 / K), w / safe)

    return pl.pallas_call(
        body,
        grid=(W2D_ROWS // BR,),
        in_specs=[
            pl.BlockSpec((BR, 128), lambda i: (i, 0)),
            pl.BlockSpec((128, 128), lambda i: (0, 0)),
        ],
        out_specs=pl.BlockSpec((BR, 128), lambda i: (i, 0)),
        out_shape=jax.ShapeDtypeStruct((W2D_ROWS, 128), jnp.float32),
    )(w2d, Q)


def _tc_linear_layernorm(agg, W, b, gamma, beta):
    BR = 1024

    def body(x_ref, w_ref, b_ref, g_ref, be_ref, o_ref):
        x = x_ref[...]
        y = lax.dot_general(x, w_ref[...], (((1,), (1,)), ((), ())),
                            preferred_element_type=jnp.float32)
        y = y + b_ref[...]
        m = jnp.mean(y, axis=-1, keepdims=True)
        dlt = y - m
        var = jnp.mean(dlt * dlt, axis=-1, keepdims=True)
        o_ref[...] = (dlt * lax.rsqrt(var + 1e-5)) * g_ref[...] + be_ref[...]

    return pl.pallas_call(
        body,
        grid=(NPAD // BR,),
        in_specs=[
            pl.BlockSpec((BR, D), lambda i: (i, 0)),
            pl.BlockSpec((D, D), lambda i: (0, 0)),
            pl.BlockSpec((1, D), lambda i: (0, 0)),
            pl.BlockSpec((1, D), lambda i: (0, 0)),
            pl.BlockSpec((1, D), lambda i: (0, 0)),
        ],
        out_specs=pl.BlockSpec((BR, D), lambda i: (i, 0)),
        out_shape=jax.ShapeDtypeStruct((NPAD, D), jnp.float32),
    )(agg, W, b.reshape(1, D), gamma.reshape(1, D), beta.reshape(1, D))


def kernel(features, neighbors, importance_weights, W, b, gamma, beta):
    idx = neighbors.astype(jnp.int32)
    pad = NPAD - N
    idx_p = jnp.pad(idx, ((0, pad), (0, 0)))
    w_p = jnp.pad(importance_weights, ((0, pad), (0, 0)), constant_values=1.0)
    blk = jnp.arange(128, dtype=jnp.int32) // K
    Q = (blk[:, None] == blk[None, :]).astype(jnp.float32)
    w_norm = _tc_normalize_weights(w_p.reshape(W2D_ROWS, 128), Q)
    agg = _sc_aggregate(features, idx_p.reshape(-1),
                        w_norm.reshape(-1))
    out = _tc_linear_layernorm(agg, W, b, gamma, beta)
    return out[:N]
